# baseline (speedup 1.0000x reference)
import functools

import jax
import jax.numpy as jnp
from jax import lax
from jax.experimental import pallas as pl
from jax.experimental.pallas import tpu as pltpu
from jax.experimental.pallas import tpu_sc as plsc

_TEMP_MIN = 2e-05
_NEG_INF = float("-inf")
_B, _V = 64, 100000
_NC, _NS, _L = 2, 16, 16
_NW = _NC * _NS
_ROWS_PER_W = _B // _NW
_U = 10
_NITER = _V // (_L * _U)

_LN2 = 0.6931471805599453
_SQRT2 = 1.4142135381698608


def _allreduce(v, op):
    for sh in (8, 4, 2, 1):
        perm = lax.iota(jnp.int32, _L) ^ sh
        g = lax.gather(
            v, perm[:, None],
            lax.GatherDimensionNumbers(offset_dims=(),
                                       collapsed_slice_dims=(0,),
                                       start_index_map=(0,)),
            slice_sizes=(1,),
            mode=lax.GatherScatterMode.PROMISE_IN_BOUNDS)
        v = op(v, g)
    return v


def _logv(v):
    bits = lax.bitcast_convert_type(v, jnp.int32)
    ex = jnp.right_shift(bits, 23) & 255
    f = lax.bitcast_convert_type((bits & 0x007FFFFF) | 0x3F800000,
                                 jnp.float32)
    big = f > _SQRT2
    f = jnp.where(big, f * 0.5, f)
    k = (ex - 127 + jnp.where(big, 1, 0)).astype(jnp.float32)
    u = (f - 1.0) / (f + 1.0)
    w = u * u
    lf = u * (2.0 + w * (0.66666666666 + w * (0.4 + w * 0.2857142857)))
    return k * _LN2 + lf


def _body(x_hbm, p_hbm, out_hbm, samp_hbm, xv, pv, sv):
    wid = lax.axis_index("s") * _NC + lax.axis_index("c")

    for r2 in range(_ROWS_PER_W):
        row = wid * _ROWS_PER_W + r2
        pltpu.sync_copy(p_hbm.at[row], pv)
        tv = jnp.maximum(pv[pl.ds(0, _L)], _TEMP_MIN)
        rtv = 1.0 / tv
        minpv = pv[pl.ds(_L, _L)]
        epsv = pv[pl.ds(2 * _L, _L)]
        etav = pv[pl.ds(3 * _L, _L)]
        lminpv = jnp.where(minpv <= 1e-37, -1e30,
                           _logv(jnp.maximum(minpv, 1e-37)))

        pltpu.sync_copy(x_hbm.at[row], xv)

        def pa(i, acc):
            for j in range(_U):
                acc = jnp.maximum(acc, xv[pl.ds((i * _U + j) * _L, _L)])
            return acc
        xmax = _allreduce(lax.fori_loop(0, _NITER, pa,
                                        jnp.full((_L,), _NEG_INF)),
                          jnp.maximum)

        def pb(i, carry):
            z2a, topa = carry
            for j in range(_U):
                off = (i * _U + j) * _L
                sm = (xv[pl.ds(off, _L)] - xmax) * rtv
                xv[pl.ds(off, _L)] = sm
                e = jnp.exp(sm)
                z2a = z2a + jnp.where(sm >= lminpv, e, 0.0)
                idx = lax.iota(jnp.int32, _L) + off
                topa = jnp.minimum(topa, jnp.where(e == 1.0, idx, _V))
            return z2a, topa
        z2a, topa = lax.fori_loop(
            0, _NITER, pb,
            (jnp.zeros((_L,), jnp.float32), jnp.full((_L,), _V, jnp.int32)))
        z2 = _allreduce(z2a, jnp.add)
        top = _allreduce(topa, jnp.minimum)

        lthr2 = jnp.maximum(lminpv, _logv(epsv * z2))

        def pc(i, carry):
            z3a, u3a = carry
            for j in range(_U):
                sm = xv[pl.ds((i * _U + j) * _L, _L)]
                z3c = jnp.where(sm >= lthr2, jnp.exp(sm), 0.0)
                z3a = z3a + z3c
                u3a = u3a + z3c * sm
            return z3a, u3a
        z3a, u3a = lax.fori_loop(
            0, _NITER, pc,
            (jnp.zeros((_L,), jnp.float32), jnp.zeros((_L,), jnp.float32)))
        z3 = _allreduce(z3a, jnp.add) + jnp.where(lthr2 <= 0.0, 0.0, 1.0)
        u3 = _allreduce(u3a, jnp.add)

        neg_ent = u3 / z3 - _logv(z3)
        sqrt_eta = jnp.exp(0.5 * _logv(etav))
        eps_eta = jnp.minimum(etav, sqrt_eta * jnp.exp(neg_ent))
        lthr3 = jnp.maximum(lthr2, _logv(eps_eta * z3))

        def pd(i, z4a):
            for j in range(_U):
                sm = xv[pl.ds((i * _U + j) * _L, _L)]
                z4a = z4a + jnp.where(sm >= lthr3, jnp.exp(sm), 0.0)
            return z4a
        z4a = lax.fori_loop(0, _NITER, pd, jnp.zeros((_L,), jnp.float32))
        z4 = _allreduce(z4a, jnp.add) + jnp.where(lthr3 <= 0.0, 0.0, 1.0)
        lz4 = _logv(z4)
        lthr3c = jnp.minimum(lthr3, 0.0)

        def pe(i, c):
            for j in range(_U):
                off = (i * _U + j) * _L
                sm = xv[pl.ds(off, _L)]
                xv[pl.ds(off, _L)] = jnp.where(sm >= lthr3c, sm - lz4,
                                               _NEG_INF)
            return c
        lax.fori_loop(0, _NITER, pe, jnp.int32(0))
        pltpu.sync_copy(xv, out_hbm.at[row])

        sv[...] = top
        pltpu.sync_copy(sv, samp_hbm.at[row])


def kernel(logits, temperature, min_p, epsilon_cutoff, eta_cutoff):
    B, V = logits.shape
    params = jnp.concatenate(
        [jnp.broadcast_to(p[:, None], (B, _L))
         for p in (temperature, min_p, epsilon_cutoff, eta_cutoff)], axis=1)
    mesh = plsc.VectorSubcoreMesh(core_axis_name="c", subcore_axis_name="s")
    f = functools.partial(
        pl.kernel, mesh=mesh,
        out_type=[jax.ShapeDtypeStruct((B, V), jnp.float32),
                  jax.ShapeDtypeStruct((B, _L), jnp.int32)],
        scratch_types=[pltpu.VMEM((V,), jnp.float32),
                       pltpu.VMEM((4 * _L,), jnp.float32),
                       pltpu.VMEM((_L,), jnp.int32)],
    )(_body)
    lp, samp = f(logits, params)
    return lp, samp[:, 0]
